# baseline (speedup 1.0000x reference)
import functools

import jax
import jax.numpy as jnp
from jax import lax
from jax.experimental import pallas as pl
from jax.experimental.pallas import tpu as pltpu
from jax.experimental.pallas import tpu_sc as plsc

N = 10000
E = 320000
D = 128

NC = 2
NS = 16
NW = NC * NS
CH = 128
G = 8
NG = 10
CPW = NG * G
LA = 1
DS = 1
NB = LA + DS
E_PAD = NW * CPW * CH
N_PAD = 10112
ZR = N_PAD // NS
EV = E // NW // 16

_mesh = plsc.VectorSubcoreMesh(core_axis_name="c", subcore_axis_name="s")


@functools.partial(
    pl.kernel,
    out_type=jax.ShapeDtypeStruct((NW, N_PAD), jnp.float32),
    mesh=_mesh,
    scratch_types=[
        pltpu.VMEM((EV, 16), jnp.int32),
        pltpu.VMEM((N_PAD,), jnp.float32),
    ],
    compiler_params=pltpu.CompilerParams(needs_layout_passes=False),
)
def _deg_kernel(dst_hbm, out_hbm, idx_v, cnt_v):
    c = lax.axis_index("c")
    s = lax.axis_index("s")
    w = c * NS + s
    pltpu.sync_copy(dst_hbm.at[w], idx_v)

    zeros16 = jnp.zeros((16,), jnp.float32)

    def zbody(i, carry):
        cnt_v[pl.ds(i * 16, 16)] = zeros16
        return carry

    lax.fori_loop(0, N_PAD // 16, zbody, 0)

    ones16 = jnp.ones((16,), jnp.float32)

    def body(i, carry):
        plsc.addupdate_scatter(cnt_v, [idx_v[i]], ones16)
        return carry

    lax.fori_loop(0, EV, body, 0)
    pltpu.sync_copy(cnt_v, out_hbm.at[w])


@functools.partial(
    pl.kernel,
    out_type=jax.ShapeDtypeStruct((NC, N_PAD, D), jnp.float32),
    mesh=_mesh,
    scratch_types=[
        pltpu.VMEM((2, G, CH), jnp.int32),
        pltpu.VMEM((2, G, CH), jnp.int32),
        pltpu.VMEM((NB, CH, D), jnp.float32),
        pltpu.VMEM_SHARED((N_PAD, D), jnp.float32),
        pltpu.SemaphoreType.DMA((NB,)),
        pltpu.SemaphoreType.DMA((NB,)),
        pltpu.SemaphoreType.DMA,
    ],
)
def _scatter_kernel(y_hbm, src_hbm, dst_hbm, zeros_hbm, out_hbm,
                    sidx, didx, rows, acc, sem_g, sem_s, sem_idx):
    c = lax.axis_index("c")
    s = lax.axis_index("s")
    w = c * NS + s
    pltpu.sync_copy(zeros_hbm.at[pl.ds(s * ZR, ZR)], acc.at[pl.ds(s * ZR, ZR)])

    def stage(g):
        gb = lax.rem(g, 2)
        pltpu.async_copy(src_hbm.at[w, g], sidx.at[gb], sem_idx)
        pltpu.async_copy(dst_hbm.at[w, g], didx.at[gb], sem_idx)

    def stage_wait(n):
        for _ in range(n):
            pltpu.make_async_copy(src_hbm.at[w, 0], sidx.at[0], sem_idx).wait()

    def fire_g(j):
        rb = lax.rem(j, NB)
        gb = lax.rem(j // G, 2)
        pltpu.async_copy(y_hbm.at[sidx.at[gb, lax.rem(j, G)]], rows.at[rb],
                         sem_g.at[rb])

    def wait_g(rb):
        pltpu.make_async_copy(y_hbm.at[sidx.at[0, 0]], rows.at[rb],
                              sem_g.at[rb]).wait()

    def fire_s(j):
        rb = lax.rem(j, NB)
        gb = lax.rem(j // G, 2)
        pltpu.async_copy(rows.at[rb], acc.at[didx.at[gb, lax.rem(j, G)]],
                         sem_s.at[rb], add=True)

    def wait_s(rb):
        pltpu.make_async_copy(rows.at[rb], acc.at[didx.at[0, 0]],
                              sem_s.at[rb]).wait()

    stage(0)
    stage(1)
    stage_wait(4)
    plsc.subcore_barrier()
    for t in range(LA):
        fire_g(t)

    def body(j, carry):
        rb = lax.rem(j, NB)
        wait_g(rb)
        fire_s(j)

        @pl.when(j <= CPW - 1 - LA)
        def _():
            @pl.when(j >= DS)
            def _():
                wait_s(lax.rem(j - DS, NB))

            @pl.when(jnp.logical_and(lax.rem(j - (DS - 1), G) == 0,
                                     jnp.logical_and(j >= G + DS - 1,
                                                     j // G + 1 < NG)))
            def _():
                stage(j // G + 1)

            jn = j + LA

            @pl.when(jnp.logical_and(lax.rem(jn, G) == 0, jn >= 2 * G))
            def _():
                stage_wait(2)

            fire_g(jn)

        return carry

    lax.fori_loop(0, CPW, body, 0)
    for t in range(NB):
        wait_s((CPW - NB + t) % NB)

    plsc.subcore_barrier()
    pltpu.sync_copy(acc.at[pl.ds(s * ZR, ZR)], out_hbm.at[c].at[pl.ds(s * ZR, ZR)])


_BR = 1000
_GRID = N // _BR


def _dred_body(deg_ref, o_ref):
    cnt = jnp.sum(deg_ref[...], axis=0)
    o_ref[...] = lax.rsqrt(cnt + 1.0)[:, None]


def _dreduce(deg):
    return pl.pallas_call(
        _dred_body,
        grid=(1,),
        in_specs=[pl.BlockSpec((NW, N_PAD), lambda i: (0, 0))],
        out_specs=pl.BlockSpec((N_PAD, 1), lambda i: (0, 0)),
        out_shape=jax.ShapeDtypeStruct((N_PAD, 1), jnp.float32),
    )(deg)


def _mms_body(x_ref, w_ref, d_ref, o_ref):
    y = jnp.dot(x_ref[...], w_ref[...], preferred_element_type=jnp.float32)
    o_ref[...] = y * d_ref[...]


def _matmul_scale(x, w, d):
    return pl.pallas_call(
        _mms_body,
        grid=(_GRID,),
        in_specs=[
            pl.BlockSpec((_BR, D), lambda i: (i, 0)),
            pl.BlockSpec((D, D), lambda i: (0, 0)),
            pl.BlockSpec((_BR, 1), lambda i: (i, 0)),
        ],
        out_specs=pl.BlockSpec((_BR, D), lambda i: (i, 0)),
        out_shape=jax.ShapeDtypeStruct((N, D), jnp.float32),
    )(x, w, d)


def _comb_body(part_ref, ys_ref, d_ref, b_ref, flag_ref, o_ref):
    sfull = part_ref[0] + part_ref[1] + ys_ref[...]
    v = d_ref[...] * sfull + b_ref[...]
    o_ref[...] = jnp.where(flag_ref[0, 0] > 0.0, jnp.maximum(v, 0.0), v)


def _combine(part, ys, d, b, flag):
    return pl.pallas_call(
        _comb_body,
        grid=(_GRID,),
        in_specs=[
            pl.BlockSpec((NC, _BR, D), lambda i: (0, i, 0)),
            pl.BlockSpec((_BR, D), lambda i: (i, 0)),
            pl.BlockSpec((_BR, 1), lambda i: (i, 0)),
            pl.BlockSpec((1, D), lambda i: (0, 0)),
            pl.BlockSpec((1, 1), lambda i: (0, 0)),
        ],
        out_specs=pl.BlockSpec((_BR, D), lambda i: (i, 0)),
        out_shape=jax.ShapeDtypeStruct((N, D), jnp.float32),
    )(part, ys, d, b, flag)


@jax.jit
def kernel(x, edge_index, W1, b1, W2, b2):
    ei = edge_index.astype(jnp.int32)
    pad = E_PAD - E
    src = jnp.concatenate([ei[0], jnp.zeros((pad,), jnp.int32)])
    dst = jnp.concatenate([ei[1], jnp.full((pad,), N, jnp.int32)])
    src = src.reshape(NW, NG, G, CH)
    dst = dst.reshape(NW, NG, G, CH)
    dst16 = ei[1].reshape(NW, EV, 16)

    zerosD = jnp.zeros((N_PAD, D), jnp.float32)

    deg = _deg_kernel(dst16)
    d = _dreduce(deg)

    W_all = jnp.stack([W1, W2])
    b_all = jnp.stack([b1.reshape(1, D), b2.reshape(1, D)])
    flags = jnp.array([[[1.0]], [[0.0]]], jnp.float32)

    def layer(carry, ops):
        w, b, flag = ops
        ys = _matmul_scale(carry, w, d)
        part = _scatter_kernel(ys, src, dst, zerosD)
        out = _combine(part, ys, d, b, flag)
        return out, None

    out, _ = lax.scan(layer, x, (W_all, b_all, flags))
    return out


# speedup vs baseline: 1.9768x; 1.9768x over previous
import functools

import jax
import jax.numpy as jnp
from jax import lax
from jax.experimental import pallas as pl
from jax.experimental.pallas import tpu as pltpu
from jax.experimental.pallas import tpu_sc as plsc

N = 10000
E = 320000
D = 128

NC = 2
NS = 16
NW = NC * NS
CH = 112
G = 6
NG = 15
CPW = NG * G
LA = 2
DS = 1
NB = LA + DS
E_PAD = NW * CPW * CH
N_PAD = 10112
ZR = N_PAD // NS
EV = E // NW // 16

_mesh = plsc.VectorSubcoreMesh(core_axis_name="c", subcore_axis_name="s")


@functools.partial(
    pl.kernel,
    out_type=jax.ShapeDtypeStruct((NW, N_PAD), jnp.float32),
    mesh=_mesh,
    scratch_types=[
        pltpu.VMEM((EV, 16), jnp.int32),
        pltpu.VMEM((N_PAD,), jnp.float32),
    ],
    compiler_params=pltpu.CompilerParams(needs_layout_passes=False),
)
def _deg_kernel(dst_hbm, out_hbm, idx_v, cnt_v):
    c = lax.axis_index("c")
    s = lax.axis_index("s")
    w = c * NS + s
    pltpu.sync_copy(dst_hbm.at[w], idx_v)

    zeros16 = jnp.zeros((16,), jnp.float32)

    def zbody(i, carry):
        cnt_v[pl.ds(i * 16, 16)] = zeros16
        return carry

    lax.fori_loop(0, N_PAD // 16, zbody, 0)

    ones16 = jnp.ones((16,), jnp.float32)

    def body(i, carry):
        plsc.addupdate_scatter(cnt_v, [idx_v[i]], ones16)
        return carry

    lax.fori_loop(0, EV, body, 0)
    pltpu.sync_copy(cnt_v, out_hbm.at[w])


@functools.partial(
    pl.kernel,
    out_type=jax.ShapeDtypeStruct((NC, N_PAD, D), jnp.float32),
    mesh=_mesh,
    scratch_types=[
        pltpu.VMEM((2, G, CH), jnp.int32),
        pltpu.VMEM((2, G, CH), jnp.int32),
        pltpu.VMEM((NB, CH, D), jnp.float32),
        pltpu.VMEM_SHARED((N_PAD, D), jnp.float32),
        pltpu.SemaphoreType.DMA((NB,)),
        pltpu.SemaphoreType.DMA((NB,)),
        pltpu.SemaphoreType.DMA,
    ],
)
def _scatter_kernel(y_hbm, src_hbm, dst_hbm, zeros_hbm, out_hbm,
                    sidx, didx, rows, acc, sem_g, sem_s, sem_idx):
    c = lax.axis_index("c")
    s = lax.axis_index("s")
    w = c * NS + s
    pltpu.sync_copy(zeros_hbm.at[pl.ds(s * ZR, ZR)], acc.at[pl.ds(s * ZR, ZR)])

    def stage(g):
        gb = lax.rem(g, 2)
        pltpu.async_copy(src_hbm.at[w, g], sidx.at[gb], sem_idx)
        pltpu.async_copy(dst_hbm.at[w, g], didx.at[gb], sem_idx)

    def stage_wait(n):
        for _ in range(n):
            pltpu.make_async_copy(src_hbm.at[w, 0], sidx.at[0], sem_idx).wait()

    def fire_g(j):
        rb = lax.rem(j, NB)
        gb = lax.rem(j // G, 2)
        pltpu.async_copy(y_hbm.at[sidx.at[gb, lax.rem(j, G)]], rows.at[rb],
                         sem_g.at[rb])

    def wait_g(rb):
        pltpu.make_async_copy(y_hbm.at[sidx.at[0, 0]], rows.at[rb],
                              sem_g.at[rb]).wait()

    def fire_s(j):
        rb = lax.rem(j, NB)
        gb = lax.rem(j // G, 2)
        pltpu.async_copy(rows.at[rb], acc.at[didx.at[gb, lax.rem(j, G)]],
                         sem_s.at[rb], add=True)

    def wait_s(rb):
        pltpu.make_async_copy(rows.at[rb], acc.at[didx.at[0, 0]],
                              sem_s.at[rb]).wait()

    stage(0)
    stage(1)
    stage_wait(4)
    plsc.subcore_barrier()
    for t in range(LA):
        fire_g(t)

    def body(j, carry):
        rb = lax.rem(j, NB)
        wait_g(rb)
        fire_s(j)

        @pl.when(j <= CPW - 1 - LA)
        def _():
            @pl.when(j >= DS)
            def _():
                wait_s(lax.rem(j - DS, NB))

            @pl.when(jnp.logical_and(lax.rem(j - (DS - 1), G) == 0,
                                     jnp.logical_and(j >= G + DS - 1,
                                                     j // G + 1 < NG)))
            def _():
                stage(j // G + 1)

            jn = j + LA

            @pl.when(jnp.logical_and(lax.rem(jn, G) == 0, jn >= 2 * G))
            def _():
                stage_wait(2)

            fire_g(jn)

        return carry

    lax.fori_loop(0, CPW, body, 0)
    for t in range(NB):
        wait_s((CPW - NB + t) % NB)

    plsc.subcore_barrier()
    pltpu.sync_copy(acc.at[pl.ds(s * ZR, ZR)], out_hbm.at[c].at[pl.ds(s * ZR, ZR)])


_BR = 1000
_GRID = N // _BR


def _dred_body(deg_ref, o_ref):
    cnt = jnp.sum(deg_ref[...], axis=0)
    o_ref[...] = lax.rsqrt(cnt + 1.0)[:, None]


def _dreduce(deg):
    return pl.pallas_call(
        _dred_body,
        grid=(1,),
        in_specs=[pl.BlockSpec((NW, N_PAD), lambda i: (0, 0))],
        out_specs=pl.BlockSpec((N_PAD, 1), lambda i: (0, 0)),
        out_shape=jax.ShapeDtypeStruct((N_PAD, 1), jnp.float32),
    )(deg)


def _mms_body(x_ref, w_ref, d_ref, o_ref):
    y = jnp.dot(x_ref[...], w_ref[...], preferred_element_type=jnp.float32)
    o_ref[...] = y * d_ref[...]


def _matmul_scale(x, w, d):
    return pl.pallas_call(
        _mms_body,
        grid=(_GRID,),
        in_specs=[
            pl.BlockSpec((_BR, D), lambda i: (i, 0)),
            pl.BlockSpec((D, D), lambda i: (0, 0)),
            pl.BlockSpec((_BR, 1), lambda i: (i, 0)),
        ],
        out_specs=pl.BlockSpec((_BR, D), lambda i: (i, 0)),
        out_shape=jax.ShapeDtypeStruct((N, D), jnp.float32),
    )(x, w, d)


def _comb_body(part_ref, ys_ref, d_ref, b_ref, flag_ref, o_ref):
    sfull = part_ref[0] + part_ref[1] + ys_ref[...]
    v = d_ref[...] * sfull + b_ref[...]
    o_ref[...] = jnp.where(flag_ref[0, 0] > 0.0, jnp.maximum(v, 0.0), v)


def _combine(part, ys, d, b, flag):
    return pl.pallas_call(
        _comb_body,
        grid=(_GRID,),
        in_specs=[
            pl.BlockSpec((NC, _BR, D), lambda i: (0, i, 0)),
            pl.BlockSpec((_BR, D), lambda i: (i, 0)),
            pl.BlockSpec((_BR, 1), lambda i: (i, 0)),
            pl.BlockSpec((1, D), lambda i: (0, 0)),
            pl.BlockSpec((1, 1), lambda i: (0, 0)),
        ],
        out_specs=pl.BlockSpec((_BR, D), lambda i: (i, 0)),
        out_shape=jax.ShapeDtypeStruct((N, D), jnp.float32),
    )(part, ys, d, b, flag)


@jax.jit
def kernel(x, edge_index, W1, b1, W2, b2):
    ei = edge_index.astype(jnp.int32)
    pad = E_PAD - E
    src = jnp.concatenate([ei[0], jnp.zeros((pad,), jnp.int32)])
    dst = jnp.concatenate([ei[1], jnp.full((pad,), N, jnp.int32)])
    src = src.reshape(NW, NG, G, CH)
    dst = dst.reshape(NW, NG, G, CH)
    dst16 = ei[1].reshape(NW, EV, 16)

    zerosD = jnp.zeros((N_PAD, D), jnp.float32)

    deg = _deg_kernel(dst16)
    d = _dreduce(deg)

    W_all = jnp.stack([W1, W2])
    b_all = jnp.stack([b1.reshape(1, D), b2.reshape(1, D)])
    flags = jnp.array([[[1.0]], [[0.0]]], jnp.float32)

    def layer(carry, ops):
        w, b, flag = ops
        ys = _matmul_scale(carry, w, d)
        part = _scatter_kernel(ys, src, dst, zerosD)
        out = _combine(part, ys, d, b, flag)
        return out, None

    out, _ = lax.scan(layer, x, (W_all, b_all, flags))
    return out


# speedup vs baseline: 2.1419x; 1.0835x over previous
import functools

import jax
import jax.numpy as jnp
from jax import lax
from jax.experimental import pallas as pl
from jax.experimental.pallas import tpu as pltpu
from jax.experimental.pallas import tpu_sc as plsc

N = 10000
E = 320000
D = 128

NC = 2
NS = 16
NW = NC * NS
CH = 112
G = 6
NG0 = 20
NG1 = 10
TOTG = NS * (NG0 + NG1)
LA = 2
DS = 1
NB = LA + DS
E_PAD = TOTG * G * CH
N_PAD = 10112
ZR = N_PAD // NS
EV = E // NW // 16

_mesh = plsc.VectorSubcoreMesh(core_axis_name="c", subcore_axis_name="s")


@functools.partial(
    pl.kernel,
    out_type=jax.ShapeDtypeStruct((NW, N_PAD), jnp.float32),
    mesh=_mesh,
    scratch_types=[
        pltpu.VMEM((EV, 16), jnp.int32),
        pltpu.VMEM((N_PAD,), jnp.float32),
    ],
    compiler_params=pltpu.CompilerParams(needs_layout_passes=False),
)
def _deg_kernel(dst_hbm, out_hbm, idx_v, cnt_v):
    c = lax.axis_index("c")
    s = lax.axis_index("s")
    w = c * NS + s
    pltpu.sync_copy(dst_hbm.at[w], idx_v)

    zeros16 = jnp.zeros((16,), jnp.float32)

    def zbody(i, carry):
        cnt_v[pl.ds(i * 16, 16)] = zeros16
        return carry

    lax.fori_loop(0, N_PAD // 16, zbody, 0)

    ones16 = jnp.ones((16,), jnp.float32)

    def body(i, carry):
        plsc.addupdate_scatter(cnt_v, [idx_v[i]], ones16)
        return carry

    lax.fori_loop(0, EV, body, 0)
    pltpu.sync_copy(cnt_v, out_hbm.at[w])


@functools.partial(
    pl.kernel,
    out_type=jax.ShapeDtypeStruct((NC, N_PAD, D), jnp.float32),
    mesh=_mesh,
    scratch_types=[
        pltpu.VMEM((2, G, CH), jnp.int32),
        pltpu.VMEM((2, G, CH), jnp.int32),
        pltpu.VMEM((NB, CH, D), jnp.float32),
        pltpu.VMEM_SHARED((N_PAD, D), jnp.float32),
        pltpu.SemaphoreType.DMA((NB,)),
        pltpu.SemaphoreType.DMA((NB,)),
        pltpu.SemaphoreType.DMA,
    ],
)
def _scatter_kernel(y_hbm, src_hbm, dst_hbm, zeros_hbm, out_hbm,
                    sidx, didx, rows, acc, sem_g, sem_s, sem_idx):
    c = lax.axis_index("c")
    s = lax.axis_index("s")
    ng = lax.select(c == 0, NG0, NG1)
    cpw = ng * G
    bg = lax.select(c == 0, s * NG0, NS * NG0 + s * NG1)
    pltpu.sync_copy(zeros_hbm.at[pl.ds(s * ZR, ZR)], acc.at[pl.ds(s * ZR, ZR)])

    def stage(g):
        gb = lax.rem(g, 2)
        pltpu.async_copy(src_hbm.at[bg + g], sidx.at[gb], sem_idx)
        pltpu.async_copy(dst_hbm.at[bg + g], didx.at[gb], sem_idx)

    def stage_wait(n):
        for _ in range(n):
            pltpu.make_async_copy(src_hbm.at[bg], sidx.at[0], sem_idx).wait()

    def fire_g(j):
        rb = lax.rem(j, NB)
        gb = lax.rem(j // G, 2)
        pltpu.async_copy(y_hbm.at[sidx.at[gb, lax.rem(j, G)]], rows.at[rb],
                         sem_g.at[rb])

    def wait_g(rb):
        pltpu.make_async_copy(y_hbm.at[sidx.at[0, 0]], rows.at[rb],
                              sem_g.at[rb]).wait()

    def fire_s(j):
        rb = lax.rem(j, NB)
        gb = lax.rem(j // G, 2)
        pltpu.async_copy(rows.at[rb], acc.at[didx.at[gb, lax.rem(j, G)]],
                         sem_s.at[rb], add=True)

    def wait_s(rb):
        pltpu.make_async_copy(rows.at[rb], acc.at[didx.at[0, 0]],
                              sem_s.at[rb]).wait()

    stage(0)
    stage(1)
    stage_wait(4)
    plsc.subcore_barrier()
    for t in range(LA):
        fire_g(t)

    def body(j, carry):
        rb = lax.rem(j, NB)
        wait_g(rb)
        fire_s(j)

        @pl.when(j <= cpw - 1 - LA)
        def _():
            @pl.when(j >= DS)
            def _():
                wait_s(lax.rem(j - DS, NB))

            @pl.when(jnp.logical_and(lax.rem(j - (DS - 1), G) == 0,
                                     jnp.logical_and(j >= G + DS - 1,
                                                     j // G + 1 < ng)))
            def _():
                stage(j // G + 1)

            jn = j + LA

            @pl.when(jnp.logical_and(lax.rem(jn, G) == 0, jn >= 2 * G))
            def _():
                stage_wait(2)

            fire_g(jn)

        return carry

    lax.fori_loop(0, cpw, body, 0)
    for t in range(NB):
        wait_s(lax.rem(cpw - NB + t, NB))

    plsc.subcore_barrier()
    pltpu.sync_copy(acc.at[pl.ds(s * ZR, ZR)], out_hbm.at[c].at[pl.ds(s * ZR, ZR)])


_BR = 1000
_GRID = N // _BR


def _dred_body(deg_ref, o_ref):
    cnt = jnp.sum(deg_ref[...], axis=0)
    o_ref[...] = lax.rsqrt(cnt + 1.0)[:, None]


def _dreduce(deg):
    return pl.pallas_call(
        _dred_body,
        grid=(1,),
        in_specs=[pl.BlockSpec((NW, N_PAD), lambda i: (0, 0))],
        out_specs=pl.BlockSpec((N_PAD, 1), lambda i: (0, 0)),
        out_shape=jax.ShapeDtypeStruct((N_PAD, 1), jnp.float32),
    )(deg)


def _mms_body(x_ref, w_ref, d_ref, o_ref):
    y = jnp.dot(x_ref[...], w_ref[...], preferred_element_type=jnp.float32)
    o_ref[...] = y * d_ref[...]


def _matmul_scale(x, w, d):
    return pl.pallas_call(
        _mms_body,
        grid=(_GRID,),
        in_specs=[
            pl.BlockSpec((_BR, D), lambda i: (i, 0)),
            pl.BlockSpec((D, D), lambda i: (0, 0)),
            pl.BlockSpec((_BR, 1), lambda i: (i, 0)),
        ],
        out_specs=pl.BlockSpec((_BR, D), lambda i: (i, 0)),
        out_shape=jax.ShapeDtypeStruct((N, D), jnp.float32),
    )(x, w, d)


def _comb_body(part_ref, ys_ref, d_ref, b_ref, flag_ref, o_ref):
    sfull = part_ref[0] + part_ref[1] + ys_ref[...]
    v = d_ref[...] * sfull + b_ref[...]
    o_ref[...] = jnp.where(flag_ref[0, 0] > 0.0, jnp.maximum(v, 0.0), v)


def _combine(part, ys, d, b, flag):
    return pl.pallas_call(
        _comb_body,
        grid=(_GRID,),
        in_specs=[
            pl.BlockSpec((NC, _BR, D), lambda i: (0, i, 0)),
            pl.BlockSpec((_BR, D), lambda i: (i, 0)),
            pl.BlockSpec((_BR, 1), lambda i: (i, 0)),
            pl.BlockSpec((1, D), lambda i: (0, 0)),
            pl.BlockSpec((1, 1), lambda i: (0, 0)),
        ],
        out_specs=pl.BlockSpec((_BR, D), lambda i: (i, 0)),
        out_shape=jax.ShapeDtypeStruct((N, D), jnp.float32),
    )(part, ys, d, b, flag)


@jax.jit
def kernel(x, edge_index, W1, b1, W2, b2):
    ei = edge_index.astype(jnp.int32)
    pad = E_PAD - E
    src = jnp.concatenate([ei[0], jnp.zeros((pad,), jnp.int32)])
    dst = jnp.concatenate([ei[1], jnp.full((pad,), N, jnp.int32)])
    src = src.reshape(TOTG, G, CH)
    dst = dst.reshape(TOTG, G, CH)
    dst16 = ei[1].reshape(NW, EV, 16)

    zerosD = jnp.zeros((N_PAD, D), jnp.float32)

    deg = _deg_kernel(dst16)
    d = _dreduce(deg)

    W_all = jnp.stack([W1, W2])
    b_all = jnp.stack([b1.reshape(1, D), b2.reshape(1, D)])
    flags = jnp.array([[[1.0]], [[0.0]]], jnp.float32)

    def layer(carry, ops):
        w, b, flag = ops
        ys = _matmul_scale(carry, w, d)
        part = _scatter_kernel(ys, src, dst, zerosD)
        out = _combine(part, ys, d, b, flag)
        return out, None

    out, _ = lax.scan(layer, x, (W_all, b_all, flags))
    return out
